# baseline (speedup 1.0000x reference)
import jax
import jax.numpy as jnp
from jax.experimental import pallas as pl
from jax.experimental.pallas import tpu as pltpu

_TOTAL_HEADS = 32


def _body(licp_ref, xk_ref, xv_ref, kc_ref, vc_ref, vl_ref,
          ko_ref, vo_ref, keys_ref, vals_ref):
    bs = ko_ref.shape[2]
    insert = xk_ref.shape[1]
    heads = ko_ref.shape[3]
    rep = _TOTAL_HEADS // heads
    li = licp_ref[0]
    cp = licp_ref[1]
    start = pl.program_id(1) * bs

    ko_ref[...] = kc_ref[...]
    vo_ref[...] = vc_ref[...]
    for i in range(insert):
        lr = cp + i - start
        @pl.when((lr >= 0) & (lr < bs))
        def _():
            ko_ref[li, 0, lr] = xk_ref[0, i]
            vo_ref[li, 0, lr] = xv_ref[0, i]

    for i in range(insert):
        lr = cp + i - start
        @pl.when((lr >= 0) & (lr < bs))
        def _():
            vl_ref[0, 0, lr] = xv_ref[0, i]
    kl = ko_ref[li, 0]
    vl = vl_ref[0, 0]
    for h in range(heads):
        keys_ref[0, :, h * rep:(h + 1) * rep, :] = jnp.broadcast_to(
            kl[:, h:h + 1, :], (bs, rep, kl.shape[2]))
        vals_ref[0, :, h * rep:(h + 1) * rep, :] = jnp.broadcast_to(
            vl[:, h:h + 1, :], (bs, rep, vl.shape[2]))


def kernel(xk, xv, k_cache, v_cache, layer_idx, cur_pos, n_rep):
    L, B, S, H, D = k_cache.shape
    insert = xk.shape[1]
    bs = 512
    li = jnp.clip(jnp.asarray(layer_idx, jnp.int32), 0, L - 1)
    cp = jnp.clip(jnp.asarray(cur_pos, jnp.int32), 0, S - insert)
    licp2 = jnp.stack([li, cp])

    grid = (B, S // bs)
    cache_spec = pl.BlockSpec((L, 1, bs, H, D), lambda b, s, ref: (0, b, s, 0, 0))
    x_spec = pl.BlockSpec((1, insert, H, D), lambda b, s, ref: (b, 0, 0, 0))
    out_spec = pl.BlockSpec((1, bs, _TOTAL_HEADS, D), lambda b, s, ref: (b, s, 0, 0))

    grid_spec = pltpu.PrefetchScalarGridSpec(
        num_scalar_prefetch=1,
        grid=grid,
        in_specs=[x_spec, x_spec, cache_spec, cache_spec,
                  pl.BlockSpec((1, 1, bs, H, D),
                               lambda b, s, ref: (ref[0], b, s, 0, 0))],
        out_specs=[cache_spec, cache_spec, out_spec, out_spec],
    )
    ko, vo, keys, values = pl.pallas_call(
        _body,
        grid_spec=grid_spec,
        out_shape=[
            jax.ShapeDtypeStruct(k_cache.shape, k_cache.dtype),
            jax.ShapeDtypeStruct(v_cache.shape, v_cache.dtype),
            jax.ShapeDtypeStruct((B, S, _TOTAL_HEADS, D), xk.dtype),
            jax.ShapeDtypeStruct((B, S, _TOTAL_HEADS, D), xv.dtype),
        ],
        compiler_params=pltpu.CompilerParams(
            dimension_semantics=("parallel", "parallel"),
        ),
    )(licp2, xk, xv, k_cache, v_cache, v_cache)
    return keys, values, ko, vo
